# baseline (speedup 1.0000x reference)
import functools

import jax
import jax.numpy as jnp
from jax import lax
from jax.experimental import pallas as pl
from jax.experimental.pallas import tpu as pltpu
from jax.experimental.pallas import tpu_sc as plsc

BATCH = 16384
HIDDEN = 64
NUM_CORES = 2
NUM_SUBCORES = 16
NUM_WORKERS = NUM_CORES * NUM_SUBCORES
B_PER_W = BATCH // NUM_WORKERS
CHUNK = 128
NCHUNK = B_PER_W // CHUNK

_mesh = plsc.VectorSubcoreMesh(core_axis_name="c", subcore_axis_name="s")


@functools.partial(
    pl.kernel,
    mesh=_mesh,
    out_type=jax.ShapeDtypeStruct((BATCH, HIDDEN), jnp.float32),
    scratch_types=[
        pltpu.VMEM((NCHUNK, CHUNK), jnp.int32),
        pltpu.VMEM((B_PER_W, HIDDEN), jnp.float32),
        pltpu.SemaphoreType.DMA,
    ],
    compiler_params=pltpu.CompilerParams(use_tc_tiling_on_sc=False),
)
def _sc_gather(labels_hbm, table_hbm, out_hbm, idx_v, rows_v, sem):
    wid = lax.axis_index("s") * NUM_CORES + lax.axis_index("c")
    base = wid * B_PER_W
    pltpu.sync_copy(labels_hbm.at[wid], idx_v)
    copies = []
    for j in range(NCHUNK):
        copies.append(
            pltpu.async_copy(
                table_hbm.at[idx_v.at[j]],
                rows_v.at[pl.ds(j * CHUNK, CHUNK)],
                sem,
            )
        )
    for c in copies:
        c.wait()
    pltpu.sync_copy(rows_v, out_hbm.at[pl.ds(base, B_PER_W)])


def kernel(labels, embedding_table):
    labels3d = labels.astype(jnp.int32).reshape(NUM_WORKERS, NCHUNK, CHUNK)
    return _sc_gather(labels3d, embedding_table)


# speedup vs baseline: 2.3444x; 2.3444x over previous
import functools

import jax
import jax.numpy as jnp
from jax import lax
from jax.experimental import pallas as pl
from jax.experimental.pallas import tpu as pltpu
from jax.experimental.pallas import tpu_sc as plsc

NUM_CLASSES = 1_000_000
HIDDEN = 64
BATCH = 16384
NUM_CORES = 2
NUM_SUBCORES = 16
NUM_WORKERS = NUM_CORES * NUM_SUBCORES
B_PER_W = BATCH // NUM_WORKERS
CH = 8
NCH = B_PER_W // CH

_mesh = plsc.VectorSubcoreMesh(core_axis_name="c", subcore_axis_name="s")


@functools.partial(
    pl.kernel,
    mesh=_mesh,
    out_type=jax.ShapeDtypeStruct((8, 8, BATCH), jnp.float32),
    scratch_types=[
        pltpu.VMEM((B_PER_W + 16,), jnp.int32),
        pltpu.VMEM((CH, 8, 8, 128), jnp.float32),
        pltpu.VMEM((8, 8, B_PER_W), jnp.float32),
        pltpu.SemaphoreType.DMA,
    ],
    compiler_params=pltpu.CompilerParams(needs_layout_passes=False),
)
def _sc_embed_t(labels_hbm, table_hbm, outt_hbm, lab_v, col_v, stage_v, sem):
    wid = lax.axis_index("s") * NUM_CORES + lax.axis_index("c")
    base = pl.multiple_of(wid * B_PER_W, B_PER_W)
    pltpu.sync_copy(labels_hbm.at[pl.ds(base, B_PER_W)], lab_v.at[pl.ds(0, B_PER_W)])
    lanes = lax.iota(jnp.int32, 16)
    r_ids = [(c * 16 + lanes) >> 3 for c in range(4)]
    h8_ids = [(c * 16 + lanes) & 7 for c in range(4)]

    def step(g, carry):
        vec = lab_v[pl.ds(g * CH, 16)]
        copies = []
        for j in range(CH):
            l_blk = pl.multiple_of((vec[j] >> 7) << 7, 128)
            copies.append(
                pltpu.async_copy(
                    table_hbm.at[:, :, pl.ds(l_blk, 128)], col_v.at[j], sem
                )
            )
        for c in copies:
            c.wait()
        for j in range(CH):
            m_sp = jnp.broadcast_to(vec[j] & 127, (16,))
            j_sp = jnp.broadcast_to(jnp.int32(j), (16,))
            col_sp = jnp.broadcast_to(g * CH + j, (16,)).astype(jnp.int32)
            for c in range(4):
                vals = plsc.load_gather(col_v, [j_sp, r_ids[c], h8_ids[c], m_sp])
                plsc.store_scatter(stage_v, [r_ids[c], h8_ids[c], col_sp], vals)
        return carry

    lax.fori_loop(0, NCH, step, 0)
    pltpu.sync_copy(stage_v, outt_hbm.at[:, :, pl.ds(base, B_PER_W)])


def kernel(labels, embedding_table):
    table3 = embedding_table.T.reshape(8, 8, NUM_CLASSES)
    outt = _sc_embed_t(labels.astype(jnp.int32), table3)
    return outt.reshape(HIDDEN, BATCH).T


# speedup vs baseline: 3.2388x; 1.3815x over previous
import functools

import jax
import jax.numpy as jnp
from jax import lax
from jax.experimental import pallas as pl
from jax.experimental.pallas import tpu as pltpu
from jax.experimental.pallas import tpu_sc as plsc

NUM_CLASSES = 1_000_000
HIDDEN = 64
BATCH = 16384
NUM_CORES = 2
NUM_SUBCORES = 16
NUM_WORKERS = NUM_CORES * NUM_SUBCORES
B_PER_W = BATCH // NUM_WORKERS
NUM_GROUPS = (NUM_CLASSES + 127) // 128
G_PER_W = (NUM_GROUPS + NUM_WORKERS - 1) // NUM_WORKERS
OWN_CAP = 784
DEPTH = 3

_mesh = plsc.VectorSubcoreMesh(core_axis_name="c", subcore_axis_name="s")


@functools.partial(
    pl.kernel,
    mesh=_mesh,
    out_type=jax.ShapeDtypeStruct((BATCH, 1, 128), jnp.float32),
    scratch_types=[
        pltpu.VMEM((OWN_CAP + 16,), jnp.int32),
        pltpu.VMEM((OWN_CAP + 16,), jnp.int32),
        pltpu.VMEM((OWN_CAP + 16,), jnp.int32),
        pltpu.VMEM((256,), jnp.int32),
        pltpu.VMEM((272,), jnp.int32),
        pltpu.VMEM((DEPTH, 8, 8, 128), jnp.float32),
        pltpu.VMEM((OWN_CAP, 1, 128), jnp.float32),
        pltpu.SemaphoreType.DMA,
        pltpu.SemaphoreType.DMA,
        pltpu.SemaphoreType.DMA,
        pltpu.SemaphoreType.DMA,
    ],
    compiler_params=pltpu.CompilerParams(needs_layout_passes=False),
)
def _sc_gather_rows(
    labels_hbm, table_hbm, rows_hbm,
    own_g, own_pv, mem_pv, bitmap, glist, col_v, rowst, s0, s1, s2, srow,
):
    wid = lax.axis_index("s") * NUM_CORES + lax.axis_index("c")
    g_lo = wid * G_PER_W
    g_hi = jnp.minimum(g_lo + G_PER_W, NUM_GROUPS)
    lanes = lax.iota(jnp.int32, 16)
    zeros16 = jnp.zeros((16,), jnp.int32)
    fsems = [s0, s1, s2]

    pltpu.sync_copy(labels_hbm, rowst.at[pl.ds(0, BATCH // 128)])
    for t in range(16):
        bitmap[pl.ds(t * 16, 16)] = zeros16

    def scan(i, cur):
        lab_f = rowst[i >> 3, 0, pl.ds((i & 7) * 16, 16)]
        lab = plsc.bitcast(lab_f, jnp.int32)
        g = lab >> 7
        mask = (g >= g_lo) & (g < g_hi)
        pos = i * 16 + lanes
        pv = (pos << 7) | (lab & 127)
        plsc.store_compressed(own_g.at[pl.ds(cur, 16)], g, mask=mask)
        plsc.store_compressed(own_pv.at[pl.ds(cur, 16)], pv, mask=mask)
        slot = jnp.clip(g - g_lo, 0, 255)
        plsc.store_scatter(bitmap, [slot], jnp.ones((16,), jnp.int32), mask=mask)
        return cur + plsc.all_reduce_population_count(mask)[0]

    cnt = lax.fori_loop(0, BATCH // 16, scan, jnp.int32(0))
    own_g[pl.ds(cnt, 16)] = jnp.full((16,), -1, jnp.int32)

    def compress(t, gcur):
        chunk = bitmap[pl.ds(t * 16, 16)]
        mask = chunk > 0
        plsc.store_compressed(glist.at[pl.ds(gcur, 16)], g_lo + t * 16 + lanes, mask=mask)
        return gcur + plsc.all_reduce_population_count(mask)[0]

    gcnt = lax.fori_loop(0, 16, compress, jnp.int32(0))
    gmax = jnp.maximum(gcnt - 1, 0)

    def fire(idx, slot):
        gi = plsc.load_gather(
            glist, [jnp.broadcast_to(jnp.minimum(idx, gmax), (16,)).astype(jnp.int32)]
        )
        gc = jnp.clip(gi[0], 0, NUM_GROUPS - 1)
        pltpu.async_copy(
            table_hbm.at[:, :, pl.ds(pl.multiple_of(gc * 128, 128), 128)],
            col_v.at[slot], fsems[slot],
        )

    for k in range(DEPTH):
        fire(jnp.int32(k), k)

    kchunks = (cnt + 15) >> 4

    def process(idx, slot, rowidx):
        pltpu.make_async_copy(
            table_hbm.at[:, :, pl.ds(0, 128)], col_v.at[slot], fsems[slot]
        ).wait()
        gi_sp = plsc.load_gather(
            glist, [jnp.broadcast_to(jnp.minimum(idx, gmax), (16,)).astype(jnp.int32)]
        )
        slot_sp = jnp.full((16,), slot, jnp.int32)

        def mscan(k, mcur):
            chunk = own_g[pl.ds(k * 16, 16)]
            mask = chunk == gi_sp
            pvc = own_pv[pl.ds(k * 16, 16)]
            plsc.store_compressed(mem_pv.at[pl.ds(mcur, 16)], pvc, mask=mask)
            return mcur + plsc.all_reduce_population_count(mask)[0]

        mcnt = lax.fori_loop(0, kchunks, mscan, jnp.int32(0))

        def member(m, ridx):
            pv_sp = plsc.load_gather(mem_pv, [jnp.broadcast_to(m, (16,)).astype(jnp.int32)])
            m_sp = pv_sp & 127
            p = jnp.clip(pv_sp[0] >> 7, 0, BATCH - 1)
            for c in range(4):
                r_ids = (c * 16 + lanes) >> 3
                h8_ids = (c * 16 + lanes) & 7
                vals = plsc.load_gather(col_v, [slot_sp, r_ids, h8_ids, m_sp])
                rowst[ridx, 0, pl.ds(c * 16, 16)] = vals
            pltpu.async_copy(rowst.at[pl.ds(ridx, 1)], rows_hbm.at[pl.ds(p, 1)], srow)
            return ridx + 1

        rowidx = lax.fori_loop(0, mcnt, member, rowidx)
        fire(idx + DEPTH, slot)
        return rowidx

    def per_round(it, carry):
        rowidx = carry
        for k in range(DEPTH):
            rowidx = process(it * DEPTH + k, k, rowidx)
        return rowidx

    nrounds = (gcnt + DEPTH - 1) // DEPTH
    total_rows = lax.fori_loop(0, nrounds, per_round, jnp.int32(0))

    for k in range(DEPTH):
        pltpu.make_async_copy(
            table_hbm.at[:, :, pl.ds(0, 128)], col_v.at[k], fsems[k]
        ).wait()

    def drain(m, carry):
        pltpu.make_async_copy(
            rows_hbm.at[pl.ds(0, 1)], rowst.at[pl.ds(0, 1)], srow
        ).wait()
        return carry

    lax.fori_loop(0, total_rows, drain, jnp.int32(0))


@functools.partial(
    pl.kernel,
    mesh=_mesh,
    out_type=jax.ShapeDtypeStruct((8, 8, BATCH), jnp.float32),
    scratch_types=[
        pltpu.VMEM((B_PER_W, 1, 128), jnp.float32),
        pltpu.VMEM((8, 8, B_PER_W), jnp.float32),
        pltpu.SemaphoreType.DMA,
    ],
    compiler_params=pltpu.CompilerParams(needs_layout_passes=False),
)
def _sc_transpose(rows_hbm, outt_hbm, loc_v, stage_v, sem):
    wid = lax.axis_index("s") * NUM_CORES + lax.axis_index("c")
    base = pl.multiple_of(wid * B_PER_W, B_PER_W)
    pltpu.sync_copy(rows_hbm.at[pl.ds(base, B_PER_W)], loc_v)
    lanes = lax.iota(jnp.int32, 16)
    z_sp = jnp.zeros((16,), jnp.int32)

    def per_h(h, carry):
        h_sp = jnp.broadcast_to(h, (16,))
        for j in range(B_PER_W // 16):
            vals = plsc.load_gather(loc_v, [j * 16 + lanes, z_sp, h_sp])
            stage_v[h >> 3, h & 7, pl.ds(j * 16, 16)] = vals
        return carry

    lax.fori_loop(0, HIDDEN, per_h, 0)
    pltpu.sync_copy(stage_v, outt_hbm.at[:, :, pl.ds(base, B_PER_W)])


def kernel(labels, embedding_table):
    table3 = embedding_table.T.reshape(8, 8, NUM_CLASSES)
    labels_f = lax.bitcast_convert_type(labels.astype(jnp.int32), jnp.float32)
    labels3 = labels_f.reshape(BATCH // 128, 1, 128)
    rows = _sc_gather_rows(labels3, table3)
    outt = _sc_transpose(rows)
    return outt.reshape(HIDDEN, BATCH).T


# speedup vs baseline: 3.4330x; 1.0600x over previous
import functools

import jax
import jax.numpy as jnp
from jax import lax
from jax.experimental import pallas as pl
from jax.experimental.pallas import tpu as pltpu
from jax.experimental.pallas import tpu_sc as plsc

NUM_CLASSES = 1_000_000
HIDDEN = 64
BATCH = 16384
NUM_CORES = 2
NUM_SUBCORES = 16
NUM_WORKERS = NUM_CORES * NUM_SUBCORES
B_PER_W = BATCH // NUM_WORKERS
NUM_GROUPS = (NUM_CLASSES + 127) // 128
G_PER_W = (NUM_GROUPS + NUM_WORKERS - 1) // NUM_WORKERS
OWN_CAP = 784
DEPTH = 3

_mesh = plsc.VectorSubcoreMesh(core_axis_name="c", subcore_axis_name="s")


@functools.partial(
    pl.kernel,
    mesh=_mesh,
    out_type=jax.ShapeDtypeStruct((BATCH, 1, 128), jnp.float32),
    scratch_types=[
        pltpu.VMEM((OWN_CAP + 16,), jnp.int32),
        pltpu.VMEM((OWN_CAP + 16,), jnp.int32),
        pltpu.VMEM((OWN_CAP + 16,), jnp.int32),
        pltpu.VMEM((256,), jnp.int32),
        pltpu.VMEM((272,), jnp.int32),
        pltpu.VMEM((DEPTH, 8, 8, 128), jnp.float32),
        pltpu.VMEM((OWN_CAP, 1, 128), jnp.float32),
        pltpu.SemaphoreType.DMA,
        pltpu.SemaphoreType.DMA,
        pltpu.SemaphoreType.DMA,
        pltpu.SemaphoreType.DMA,
    ],
    compiler_params=pltpu.CompilerParams(needs_layout_passes=False),
)
def _sc_gather_rows(
    labels_hbm, table_hbm, rows_hbm,
    own_g, own_pv, mem_pv, bitmap, glist, col_v, rowst, s0, s1, s2, srow,
):
    wid = lax.axis_index("s") * NUM_CORES + lax.axis_index("c")
    g_lo = wid * G_PER_W
    g_hi = jnp.minimum(g_lo + G_PER_W, NUM_GROUPS)
    lanes = lax.iota(jnp.int32, 16)
    zeros16 = jnp.zeros((16,), jnp.int32)
    fsems = [s0, s1, s2]

    pltpu.sync_copy(labels_hbm, rowst.at[pl.ds(0, BATCH // 128)])
    for t in range(16):
        bitmap[pl.ds(t * 16, 16)] = zeros16

    def scan(i, cur):
        lab_f = rowst[i >> 3, 0, pl.ds((i & 7) * 16, 16)]
        lab = plsc.bitcast(lab_f, jnp.int32)
        g = lab >> 7
        mask = (g >= g_lo) & (g < g_hi)
        pos = i * 16 + lanes
        pv = (pos << 7) | (lab & 127)
        plsc.store_compressed(own_g.at[pl.ds(cur, 16)], g, mask=mask)
        plsc.store_compressed(own_pv.at[pl.ds(cur, 16)], pv, mask=mask)
        slot = jnp.clip(g - g_lo, 0, 255)
        plsc.store_scatter(bitmap, [slot], jnp.ones((16,), jnp.int32), mask=mask)
        return cur + plsc.all_reduce_population_count(mask)[0]

    cnt = lax.fori_loop(0, BATCH // 16, scan, jnp.int32(0))
    own_g[pl.ds(cnt, 16)] = jnp.full((16,), -1, jnp.int32)

    def compress(t, gcur):
        chunk = bitmap[pl.ds(t * 16, 16)]
        mask = chunk > 0
        plsc.store_compressed(glist.at[pl.ds(gcur, 16)], g_lo + t * 16 + lanes, mask=mask)
        return gcur + plsc.all_reduce_population_count(mask)[0]

    gcnt = lax.fori_loop(0, 16, compress, jnp.int32(0))
    gmax = jnp.maximum(gcnt - 1, 0)

    def fire(idx, slot):
        gi = plsc.load_gather(
            glist, [jnp.broadcast_to(jnp.minimum(idx, gmax), (16,)).astype(jnp.int32)]
        )
        gc = jnp.clip(gi[0], 0, NUM_GROUPS - 1)
        pltpu.async_copy(
            table_hbm.at[:, :, pl.ds(pl.multiple_of(gc * 128, 128), 128)],
            col_v.at[slot], fsems[slot],
        )

    for k in range(DEPTH):
        fire(jnp.int32(k), k)

    kchunks = (cnt + 15) >> 4

    def process(idx, slot, rowidx):
        pltpu.make_async_copy(
            table_hbm.at[:, :, pl.ds(0, 128)], col_v.at[slot], fsems[slot]
        ).wait()
        gi_sp = plsc.load_gather(
            glist, [jnp.broadcast_to(jnp.minimum(idx, gmax), (16,)).astype(jnp.int32)]
        )
        slot_sp = jnp.full((16,), slot, jnp.int32)

        def mscan(k, mcur):
            chunk = own_g[pl.ds(k * 16, 16)]
            mask = chunk == gi_sp
            pvc = own_pv[pl.ds(k * 16, 16)]
            plsc.store_compressed(mem_pv.at[pl.ds(mcur, 16)], pvc, mask=mask)
            return mcur + plsc.all_reduce_population_count(mask)[0]

        mcnt = lax.fori_loop(0, kchunks, mscan, jnp.int32(0))

        def member(m, ridx):
            pv_sp = plsc.load_gather(mem_pv, [jnp.broadcast_to(m, (16,)).astype(jnp.int32)])
            m_sp = pv_sp & 127
            p = jnp.clip(pv_sp[0] >> 7, 0, BATCH - 1)
            for c in range(4):
                r_ids = (c * 16 + lanes) >> 3
                h8_ids = (c * 16 + lanes) & 7
                vals = plsc.load_gather(col_v, [slot_sp, r_ids, h8_ids, m_sp])
                rowst[ridx, 0, pl.ds(c * 16, 16)] = vals
            pltpu.async_copy(rowst.at[pl.ds(ridx, 1)], rows_hbm.at[pl.ds(p, 1)], srow)
            return ridx + 1

        rowidx = lax.fori_loop(0, mcnt, member, rowidx)
        fire(idx + DEPTH, slot)
        return rowidx

    def per_round(it, carry):
        rowidx = carry
        for k in range(DEPTH):
            rowidx = process(it * DEPTH + k, k, rowidx)
        return rowidx

    nrounds = (gcnt + DEPTH - 1) // DEPTH
    total_rows = lax.fori_loop(0, nrounds, per_round, jnp.int32(0))

    for k in range(DEPTH):
        pltpu.make_async_copy(
            table_hbm.at[:, :, pl.ds(0, 128)], col_v.at[k], fsems[k]
        ).wait()

    def drain(m, carry):
        pltpu.make_async_copy(
            rows_hbm.at[pl.ds(0, 1)], rowst.at[pl.ds(0, 1)], srow
        ).wait()
        return carry

    lax.fori_loop(0, total_rows, drain, jnp.int32(0))


@functools.partial(
    pl.kernel,
    mesh=_mesh,
    out_type=jax.ShapeDtypeStruct((8, 8, BATCH), jnp.float32),
    scratch_types=[
        pltpu.VMEM((B_PER_W, 1, 128), jnp.float32),
        pltpu.VMEM((8, 8, B_PER_W), jnp.float32),
        pltpu.SemaphoreType.DMA,
    ],
    compiler_params=pltpu.CompilerParams(needs_layout_passes=False),
)
def _sc_transpose(rows_hbm, outt_hbm, loc_v, stage_v, sem):
    wid = lax.axis_index("s") * NUM_CORES + lax.axis_index("c")
    base = pl.multiple_of(wid * B_PER_W, B_PER_W)
    pltpu.sync_copy(rows_hbm.at[pl.ds(base, B_PER_W)], loc_v)
    lanes = lax.iota(jnp.int32, 16)
    z_sp = jnp.zeros((16,), jnp.int32)

    def per_j(j, carry):
        pos_ids = j * 16 + lanes
        for h0 in range(0, HIDDEN, 16):
            for d in range(16):
                h_ids = h0 + ((lanes + d) & 15)
                vals = plsc.load_gather(loc_v, [pos_ids, z_sp, h_ids])
                plsc.store_scatter(stage_v, [h_ids >> 3, h_ids & 7, pos_ids], vals)
        return carry

    lax.fori_loop(0, B_PER_W // 16, per_j, 0)
    pltpu.sync_copy(stage_v, outt_hbm.at[:, :, pl.ds(base, B_PER_W)])


def kernel(labels, embedding_table):
    table3 = embedding_table.T.reshape(8, 8, NUM_CLASSES)
    labels_f = lax.bitcast_convert_type(labels.astype(jnp.int32), jnp.float32)
    labels3 = labels_f.reshape(BATCH // 128, 1, 128)
    rows = _sc_gather_rows(labels3, table3)
    outt = _sc_transpose(rows)
    return outt.reshape(HIDDEN, BATCH).T
